# baseline (speedup 1.0000x reference)
import jax
import jax.numpy as jnp
import numpy as np
from jax import lax
from jax.experimental import pallas as pl
from jax.experimental.pallas import tpu as pltpu
from jax.experimental.pallas import tpu_sc as plsc

N = 10000
E = 320000
D = 128
D_KAF = 20

NT = 32
B = 128
PER_TILE = ((E + NT - 1) // NT + 2 * B - 1) // (2 * B) * (2 * B)
NCH = PER_TILE // B
E_PAD = PER_TILE * NT
N_PAD = 10240
RPT = N_PAD // 16

_PTS = np.linspace(-3.0, 3.0, D_KAF).astype(np.float32)
_DELTA = _PTS[1] - _PTS[0]
_GAMMA = np.float32(1.0) / (np.float32(2.0) * _DELTA * _DELTA)

def _p_body(h, src3, dst3, z128, p_out, idxs, idxd, rows0, rows1, spP,
            sem0, sem1, ssem0, ssem1):
    c = lax.axis_index("c")
    s = lax.axis_index("s")
    g = c * 16 + s
    hch = NCH // 2

    pltpu.sync_copy(z128, spP.at[pl.ds(s * RPT, RPT)])
    plsc.subcore_barrier()

    for ph in range(2):
        base = ph * hch
        pltpu.sync_copy(src3.at[g, pl.ds(base, hch)], idxs)
        pltpu.sync_copy(dst3.at[g, pl.ds(base, hch)], idxd)
        pltpu.async_copy(h.at[idxs.at[0]], rows0, sem0)
        pltpu.async_copy(h.at[idxs.at[1]], rows1, sem1)

        def body(j, carry):
            i0 = 2 * j
            i1 = i0 + 1
            pltpu.make_async_copy(h.at[idxs.at[i0]], rows0, sem0).wait()
            pltpu.async_copy(rows0, spP.at[idxd.at[i0]], ssem0, add=True)
            pltpu.make_async_copy(h.at[idxs.at[i1]], rows1, sem1).wait()
            pltpu.async_copy(rows1, spP.at[idxd.at[i1]], ssem1, add=True)

            @pl.when(i0 + 2 < hch)
            def _():
                pltpu.make_async_copy(rows0, spP.at[idxd.at[i0]], ssem0).wait()
                pltpu.async_copy(h.at[idxs.at[i0 + 2]], rows0, sem0)

            @pl.when(i1 + 2 < hch)
            def _():
                pltpu.make_async_copy(rows1, spP.at[idxd.at[i1]], ssem1).wait()
                pltpu.async_copy(h.at[idxs.at[i1 + 2]], rows1, sem1)

            return carry

        lax.fori_loop(0, hch // 2, body, 0)
        pltpu.make_async_copy(rows0, spP.at[idxd.at[hch - 2]], ssem0).wait()
        pltpu.make_async_copy(rows1, spP.at[idxd.at[hch - 1]], ssem1).wait()
    plsc.subcore_barrier()

    pltpu.sync_copy(spP.at[pl.ds(s * RPT, RPT)], p_out.at[c, pl.ds(s * RPT, RPT)])


import functools


@functools.lru_cache(maxsize=None)
def _sc_kernels():
    mesh = plsc.VectorSubcoreMesh(core_axis_name="c", subcore_axis_name="s")
    p_pass = pl.kernel(
        _p_body,
        out_type=[jax.ShapeDtypeStruct((2, N_PAD, D), jnp.float32)],
        mesh=mesh,
        scratch_types=[
            pltpu.VMEM((NCH // 2, B), jnp.int32),
            pltpu.VMEM((NCH // 2, B), jnp.int32),
            pltpu.VMEM((B, D), jnp.float32),
            pltpu.VMEM((B, D), jnp.float32),
            pltpu.VMEM_SHARED((N_PAD, D), jnp.float32),
            pltpu.SemaphoreType.DMA,
            pltpu.SemaphoreType.DMA,
            pltpu.SemaphoreType.DMA,
            pltpu.SemaphoreType.DMA,
        ],
    )
    return p_pass


def _dense_body(h_ref, p_ref, aux_ref, nodew_ref, edgew_ref, nbrw_ref,
                bias_ref, alphat_ref, out_ref):
    p = p_ref[0] + p_ref[1]
    aux = aux_ref[0, :, :16] + aux_ref[1, :, :16]
    inv = 1.0 / jnp.maximum(aux[:, 2:3], 1.0)
    aggr = (jnp.dot(aux, edgew_ref[...], preferred_element_type=jnp.float32)
            + jnp.dot(p, nbrw_ref[...], preferred_element_type=jnp.float32)) * inv
    pre = (aggr + jnp.dot(h_ref[...], nodew_ref[...],
                          preferred_element_type=jnp.float32) + bias_ref[...])
    alpha = alphat_ref[...]
    acc = jnp.zeros_like(pre)
    for j in range(D_KAF):
        t = pre - _PTS[j]
        acc = acc + jnp.exp(-_GAMMA * t * t) * alpha[j]
    out_ref[...] = acc


_R = 512


def _dense(h, p, aux, node_w, edge_w16, nbr_w, bias_b, alpha_t):
    return pl.pallas_call(
        _dense_body,
        grid=(N_PAD // _R,),
        in_specs=[
            pl.BlockSpec((_R, D), lambda i: (i, 0)),
            pl.BlockSpec((2, _R, D), lambda i: (0, i, 0)),
            pl.BlockSpec((2, _R, D), lambda i: (0, i, 0)),
            pl.BlockSpec((D, D), lambda i: (0, 0)),
            pl.BlockSpec((16, D), lambda i: (0, 0)),
            pl.BlockSpec((D, D), lambda i: (0, 0)),
            pl.BlockSpec((1, D), lambda i: (0, 0)),
            pl.BlockSpec((D_KAF, D), lambda i: (0, 0)),
        ],
        out_specs=pl.BlockSpec((_R, D), lambda i: (i, 0)),
        out_shape=jax.ShapeDtypeStruct((N_PAD, D), jnp.float32),
    )(h, p, aux, node_w, edge_w16, nbr_w, bias_b, alpha_t)


def kernel(x, edge_index, edge_attr,
           node_w0, edge_w0, nbr_w0, bias0, alpha0,
           node_w1, edge_w1, nbr_w1, bias1, alpha1):
    f32 = jnp.float32
    src = edge_index[0].astype(jnp.int32)
    dst = edge_index[1].astype(jnp.int32)
    pad = E_PAD - E
    src3 = jnp.concatenate([src, jnp.zeros((pad,), jnp.int32)]).reshape(NT, NCH, B)
    dst3 = jnp.concatenate([dst, jnp.full((pad,), N, jnp.int32)]).reshape(NT, NCH, B)
    aug128 = jnp.concatenate(
        [edge_attr, jnp.ones((E, 1), f32), jnp.zeros((E, 125), f32)], axis=1)
    aug128 = jnp.concatenate([aug128, jnp.zeros((pad, D), f32)])
    ids3 = jnp.arange(E_PAD, dtype=jnp.int32).reshape(NT, NCH, B)
    z128 = jnp.zeros((RPT, D), f32)
    x_pad = jnp.concatenate([x, jnp.zeros((N_PAD - N, D), f32)])

    ew16_0 = jnp.concatenate([edge_w0, jnp.zeros((14, D), f32)])
    ew16_1 = jnp.concatenate([edge_w1, jnp.zeros((14, D), f32)])

    _sc_p_pass = _sc_kernels()
    aux0, = _sc_p_pass(aug128, ids3, dst3, z128)
    p0, = _sc_p_pass(x_pad, src3, dst3, z128)
    h1 = _dense(x_pad, p0, aux0, node_w0, ew16_0, nbr_w0,
                bias0.reshape(1, D), alpha0.T)
    p1, = _sc_p_pass(h1, src3, dst3, z128)
    h2 = _dense(h1, p1, aux0, node_w1, ew16_1, nbr_w1,
                bias1.reshape(1, D), alpha1.T)
    return h2[:N]


# speedup vs baseline: 1.2259x; 1.2259x over previous
import jax
import jax.numpy as jnp
import numpy as np
from jax import lax
from jax.experimental import pallas as pl
from jax.experimental.pallas import tpu as pltpu
from jax.experimental.pallas import tpu_sc as plsc

N = 10000
E = 320000
D = 128
D_KAF = 20

NT = 32
B = 128
PER_TILE = ((E + NT - 1) // NT + B - 1) // B * B
NCH = PER_TILE // B
E_PAD = PER_TILE * NT
N_PAD = 10240
RPT = N_PAD // 16

_PTS = np.linspace(-3.0, 3.0, D_KAF).astype(np.float32)
_DELTA = _PTS[1] - _PTS[0]
_GAMMA = np.float32(1.0) / (np.float32(2.0) * _DELTA * _DELTA)

def _p_body(h, src3, dst3, z128, p_out, idxs, idxd, rows, spP, sem):
    c = lax.axis_index("c")
    s = lax.axis_index("s")
    g = c * 16 + s

    pltpu.sync_copy(z128, spP.at[pl.ds(s * RPT, RPT)])
    pltpu.sync_copy(src3.at[g], idxs)
    pltpu.sync_copy(dst3.at[g], idxd)
    plsc.subcore_barrier()

    def chunk(i, carry):
        pltpu.async_copy(h.at[idxs.at[i]], rows, sem).wait()
        pltpu.sync_copy(rows, spP.at[idxd.at[i]], add=True)
        return carry

    lax.fori_loop(0, NCH, chunk, 0)
    plsc.subcore_barrier()

    pltpu.sync_copy(spP.at[pl.ds(s * RPT, RPT)], p_out.at[c, pl.ds(s * RPT, RPT)])


import functools


@functools.lru_cache(maxsize=None)
def _sc_kernels():
    mesh = plsc.VectorSubcoreMesh(core_axis_name="c", subcore_axis_name="s")
    p_pass = pl.kernel(
        _p_body,
        out_type=[jax.ShapeDtypeStruct((2, N_PAD, D), jnp.float32)],
        mesh=mesh,
        scratch_types=[
            pltpu.VMEM((NCH, B), jnp.int32),
            pltpu.VMEM((NCH, B), jnp.int32),
            pltpu.VMEM((B, D), jnp.float32),
            pltpu.VMEM_SHARED((N_PAD, D), jnp.float32),
            pltpu.SemaphoreType.DMA,
        ],
    )
    return p_pass


def _dense_body(h_ref, p_ref, aux_ref, nodew_ref, edgew_ref, nbrw_ref,
                bias_ref, alphat_ref, out_ref):
    p = p_ref[0] + p_ref[1]
    aux = aux_ref[0, :, :16] + aux_ref[1, :, :16]
    inv = 1.0 / jnp.maximum(aux[:, 2:3], 1.0)
    aggr = (jnp.dot(aux, edgew_ref[...], preferred_element_type=jnp.float32)
            + jnp.dot(p, nbrw_ref[...], preferred_element_type=jnp.float32)) * inv
    pre = (aggr + jnp.dot(h_ref[...], nodew_ref[...],
                          preferred_element_type=jnp.float32) + bias_ref[...])
    alpha = alphat_ref[...]
    acc = jnp.zeros_like(pre)
    for j in range(D_KAF):
        t = pre - _PTS[j]
        acc = acc + jnp.exp(-_GAMMA * t * t) * alpha[j]
    out_ref[...] = acc


_R = 512


def _dense(h, p, aux, node_w, edge_w16, nbr_w, bias_b, alpha_t):
    return pl.pallas_call(
        _dense_body,
        grid=(N_PAD // _R,),
        in_specs=[
            pl.BlockSpec((_R, D), lambda i: (i, 0)),
            pl.BlockSpec((2, _R, D), lambda i: (0, i, 0)),
            pl.BlockSpec((2, _R, D), lambda i: (0, i, 0)),
            pl.BlockSpec((D, D), lambda i: (0, 0)),
            pl.BlockSpec((16, D), lambda i: (0, 0)),
            pl.BlockSpec((D, D), lambda i: (0, 0)),
            pl.BlockSpec((1, D), lambda i: (0, 0)),
            pl.BlockSpec((D_KAF, D), lambda i: (0, 0)),
        ],
        out_specs=pl.BlockSpec((_R, D), lambda i: (i, 0)),
        out_shape=jax.ShapeDtypeStruct((N_PAD, D), jnp.float32),
    )(h, p, aux, node_w, edge_w16, nbr_w, bias_b, alpha_t)


def kernel(x, edge_index, edge_attr,
           node_w0, edge_w0, nbr_w0, bias0, alpha0,
           node_w1, edge_w1, nbr_w1, bias1, alpha1):
    f32 = jnp.float32
    src = edge_index[0].astype(jnp.int32)
    dst = edge_index[1].astype(jnp.int32)
    pad = E_PAD - E
    src3 = jnp.concatenate([src, jnp.zeros((pad,), jnp.int32)]).reshape(NT, NCH, B)
    dst3 = jnp.concatenate([dst, jnp.full((pad,), N, jnp.int32)]).reshape(NT, NCH, B)
    aug128 = jnp.concatenate(
        [edge_attr, jnp.ones((E, 1), f32), jnp.zeros((E, 125), f32)], axis=1)
    aug128 = jnp.concatenate([aug128, jnp.zeros((pad, D), f32)])
    ids3 = jnp.arange(E_PAD, dtype=jnp.int32).reshape(NT, NCH, B)
    z128 = jnp.zeros((RPT, D), f32)
    x_pad = jnp.concatenate([x, jnp.zeros((N_PAD - N, D), f32)])

    ew16_0 = jnp.concatenate([edge_w0, jnp.zeros((14, D), f32)])
    ew16_1 = jnp.concatenate([edge_w1, jnp.zeros((14, D), f32)])

    _sc_p_pass = _sc_kernels()
    aux0, = _sc_p_pass(aug128, ids3, dst3, z128)
    p0, = _sc_p_pass(x_pad, src3, dst3, z128)
    h1 = _dense(x_pad, p0, aux0, node_w0, ew16_0, nbr_w0,
                bias0.reshape(1, D), alpha0.T)
    p1, = _sc_p_pass(h1, src3, dst3, z128)
    h2 = _dense(h1, p1, aux0, node_w1, ew16_1, nbr_w1,
                bias1.reshape(1, D), alpha1.T)
    return h2[:N]
